# Initial kernel scaffold; baseline (speedup 1.0000x reference)
import jax
import jax.numpy as jnp
from jax.experimental import pallas as pl


def kernel(x, weights, input_channel_indices, input_pos_x_indices, input_pos_y_indices):
    raise NotImplementedError("write your pallas kernel here")


# speedup vs baseline: 17.4930x; 17.4930x over previous
import functools

import jax
import jax.numpy as jnp
import numpy as np
from jax.experimental import pallas as pl
from jax.experimental.pallas import tpu as pltpu

_PAD = 1
_TAU = 1.0

_GATE_COEFFS_NP = np.array([
    [0, 0, 0, 0], [0, 0, 0, 1], [0, 1, 0, -1], [0, 1, 0, 0],
    [0, 0, 1, -1], [0, 0, 1, 0], [0, 1, 1, -2], [0, 1, 1, -1],
    [1, -1, -1, 1], [1, -1, -1, 2], [1, 0, -1, 0], [1, 0, -1, 1],
    [1, -1, 0, 0], [1, -1, 0, 1], [1, 0, 0, -1], [1, 0, 0, 0],
], dtype=np.float32)


def _tree_pool_body(idx_ref, x0, x1, x2, x3, cf, er, ec, out_ref, *, H, W):
  o = pl.program_id(1)
  Hp, Wp = H + 2 * _PAD, W + 2 * _PAD
  leaves = []
  for i, xr in enumerate((x0, x1, x2, x3)):
    dy = idx_ref[1, o, i]
    dx = idx_ref[2, o, i]
    p = xr[0, 0]
    p = pltpu.roll(p, Hp - dy, axis=0)
    p = pltpu.roll(p, Wp - dx, axis=1)
    leaves.append(p[0:H, 0:W])
  l0, l1, l2, l3 = leaves
  c = [cf[0, 0, k] for k in range(12)]
  g0 = c[0] + c[1] * l0 + c[2] * l1 + c[3] * (l0 * l1)
  g1 = c[4] + c[5] * l2 + c[6] * l3 + c[7] * (l2 * l3)
  t = c[8] + c[9] * g0 + c[10] * g1 + c[11] * (g0 * g1)
  u = 1.0 - t
  m = u * jnp.roll(u, -1, axis=0)
  rp = jax.lax.dot_general(er[...], m, (((0,), (0,)), ((), ())),
                           preferred_element_type=jnp.float32)
  n = rp * jnp.roll(rp, -1, axis=1)
  cp = jnp.dot(n, ec[...], preferred_element_type=jnp.float32)
  out_ref[0, 0] = 1.0 - cp


def kernel(x, weights, input_channel_indices, input_pos_x_indices,
           input_pos_y_indices):
  B, C, H, W = x.shape
  O = weights.shape[0]
  OHp, OWp = H // 2, W // 2

  gate_coeffs = jnp.asarray(_GATE_COEFFS_NP)
  w_soft = jax.nn.softmax(weights / _TAU, axis=-1).astype(x.dtype)
  w_hard = jax.nn.one_hot(jnp.argmax(weights, axis=-1), 16, dtype=x.dtype)
  w = w_hard + w_soft - w_soft
  coeffs = jnp.einsum('ogk,kc->ogc', w, gate_coeffs)
  coeffs = coeffs.reshape(O, 1, 12)

  xp = jnp.pad(x, ((0, 0), (0, 0), (_PAD, _PAD), (_PAD, _PAD)))

  idx = jnp.stack([input_channel_indices, input_pos_y_indices,
                   input_pos_x_indices]).astype(jnp.int32)

  er = (jnp.arange(H)[:, None] == 2 * jnp.arange(OHp)[None, :]).astype(x.dtype)
  ec = (jnp.arange(W)[:, None] == 2 * jnp.arange(OWp)[None, :]).astype(x.dtype)

  def leaf_spec(i):
    return pl.BlockSpec(
        (1, 1, H + 2 * _PAD, W + 2 * _PAD),
        lambda b, o, idx_ref, i=i: (b, idx_ref[0, o, i], 0, 0))

  grid_spec = pltpu.PrefetchScalarGridSpec(
      num_scalar_prefetch=1,
      grid=(B, O),
      in_specs=[leaf_spec(0), leaf_spec(1), leaf_spec(2), leaf_spec(3),
                pl.BlockSpec((1, 1, 12), lambda b, o, idx_ref: (o, 0, 0)),
                pl.BlockSpec((H, OHp), lambda b, o, idx_ref: (0, 0)),
                pl.BlockSpec((W, OWp), lambda b, o, idx_ref: (0, 0))],
      out_specs=pl.BlockSpec((1, 1, OHp, OWp),
                             lambda b, o, idx_ref: (b, o, 0, 0)),
  )

  return pl.pallas_call(
      functools.partial(_tree_pool_body, H=H, W=W),
      grid_spec=grid_spec,
      out_shape=jax.ShapeDtypeStruct((B, O, OHp, OWp), x.dtype),
  )(idx, xp, xp, xp, xp, coeffs, er, ec)


# speedup vs baseline: 23.0433x; 1.3173x over previous
import functools

import jax
import jax.numpy as jnp
import numpy as np
from jax.experimental import pallas as pl
from jax.experimental.pallas import tpu as pltpu

_TAU = 1.0
_NLEAF = 4

_GATE_COEFFS_NP = np.array([
    [0, 0, 0, 0], [0, 0, 0, 1], [0, 1, 0, -1], [0, 1, 0, 0],
    [0, 0, 1, -1], [0, 0, 1, 0], [0, 1, 1, -2], [0, 1, 1, -1],
    [1, -1, -1, 1], [1, -1, -1, 2], [1, 0, -1, 0], [1, 0, -1, 1],
    [1, -1, 0, 0], [1, -1, 0, 1], [1, 0, 0, -1], [1, 0, 0, 0],
], dtype=np.float32)


def _body(idx_ref, x0, x1, x2, x3, cf, er, ec, out_ref, *, B, H, W):
  o = pl.program_id(0)
  rowi = jax.lax.broadcasted_iota(jnp.int32, (H, W), 0)
  coli = jax.lax.broadcasted_iota(jnp.int32, (H, W), 1)
  leaves = []
  for i, xr in enumerate((x0, x1, x2, x3)):
    dy = idx_ref[1, o, i]
    dx = idx_ref[2, o, i]
    p = xr[:, 0]
    p = pltpu.roll(p, H + 1 - dy, axis=1)
    p = pltpu.roll(p, W + 1 - dx, axis=2)
    bad = ((rowi == 0) & (dy == 0)) | ((rowi == H - 1) & (dy == 2)) \
        | ((coli == 0) & (dx == 0)) | ((coli == W - 1) & (dx == 2))
    leaves.append(jnp.where(bad[None], 0.0, p))
  l0, l1, l2, l3 = leaves
  c = [cf[0, 0, k] for k in range(12)]
  g0 = c[0] + c[1] * l0 + c[2] * l1 + c[3] * (l0 * l1)
  g1 = c[4] + c[5] * l2 + c[6] * l3 + c[7] * (l2 * l3)
  t = c[8] + c[9] * g0 + c[10] * g1 + c[11] * (g0 * g1)
  u = 1.0 - t
  for b in range(B):
    ub = u[b]
    m = ub * jnp.roll(ub, -1, axis=0)
    rp = jax.lax.dot_general(er[...], m, (((0,), (0,)), ((), ())),
                             preferred_element_type=jnp.float32)
    n = rp * jnp.roll(rp, -1, axis=1)
    cp = jnp.dot(n, ec[...], preferred_element_type=jnp.float32)
    out_ref[b, 0] = 1.0 - cp


def kernel(x, weights, input_channel_indices, input_pos_x_indices,
           input_pos_y_indices):
  B, C, H, W = x.shape
  O = weights.shape[0]
  OHp, OWp = H // 2, W // 2

  gate_coeffs = jnp.asarray(_GATE_COEFFS_NP)
  w_soft = jax.nn.softmax(weights / _TAU, axis=-1).astype(x.dtype)
  w_hard = jax.nn.one_hot(jnp.argmax(weights, axis=-1), 16, dtype=x.dtype)
  w = w_hard + w_soft - w_soft
  coeffs = jnp.einsum('ogk,kc->ogc', w, gate_coeffs)
  coeffs = coeffs.reshape(O, 1, 12)

  idx = jnp.stack([input_channel_indices, input_pos_y_indices,
                   input_pos_x_indices]).astype(jnp.int32)

  er = (jnp.arange(H)[:, None] == 2 * jnp.arange(OHp)[None, :]).astype(x.dtype)
  ec = (jnp.arange(W)[:, None] == 2 * jnp.arange(OWp)[None, :]).astype(x.dtype)

  def leaf_spec(i):
    return pl.BlockSpec(
        (B, 1, H, W), lambda o, idx_ref, i=i: (0, idx_ref[0, o, i], 0, 0))

  grid_spec = pltpu.PrefetchScalarGridSpec(
      num_scalar_prefetch=1,
      grid=(O,),
      in_specs=[leaf_spec(0), leaf_spec(1), leaf_spec(2), leaf_spec(3),
                pl.BlockSpec((1, 1, 12), lambda o, idx_ref: (o, 0, 0)),
                pl.BlockSpec((H, OHp), lambda o, idx_ref: (0, 0)),
                pl.BlockSpec((W, OWp), lambda o, idx_ref: (0, 0))],
      out_specs=pl.BlockSpec((B, 1, OHp, OWp), lambda o, idx_ref: (0, o, 0, 0)),
  )

  return pl.pallas_call(
      functools.partial(_body, B=B, H=H, W=W),
      grid_spec=grid_spec,
      out_shape=jax.ShapeDtypeStruct((B, O, OHp, OWp), x.dtype),
  )(idx, x, x, x, x, coeffs, er, ec)
